# baseline (speedup 1.0000x reference)
import jax
import jax.numpy as jnp
from jax import lax
from jax.experimental import pallas as pl
from jax.experimental.pallas import tpu as pltpu
from jax.experimental.pallas import tpu_sc as plsc

_B, _C, _H, _W, _K = 2, 96, 224, 224, 2
_CP = 128
_HW = _H * _W
_N = _B * _HW
_NC, _NS, _L = 2, 16, 16
_NW = _NC * _NS
_ROWS_PER = _N // _NW
_CHUNK = 64
_NCHUNK = _ROWS_PER // _CHUNK


def _body(inp_hbm, flow_hbm, att_hbm, out_hbm,
          fxw, fyw, avw, ia, ib, ic, id_, wa, wb, wc, wd,
          ra, rb, rc, rd, ov, sem0, sem1, sem_out):
    wid = lax.axis_index("s") * _NC + lax.axis_index("c")
    b = wid // 16
    wloc = wid - b * 16
    imgbase = b * _HW
    fbase0 = b * (2 * _K * _HW)
    abase0 = b * (_K * _HW)
    loff = wloc * _ROWS_PER

    for k in range(_K):
        pltpu.sync_copy(flow_hbm.at[pl.ds(fbase0 + 2 * k * _HW + loff, _ROWS_PER)],
                        fxw.at[pl.ds(k * _ROWS_PER, _ROWS_PER)])
        pltpu.sync_copy(flow_hbm.at[pl.ds(fbase0 + (2 * k + 1) * _HW + loff, _ROWS_PER)],
                        fyw.at[pl.ds(k * _ROWS_PER, _ROWS_PER)])
        pltpu.sync_copy(att_hbm.at[pl.ds(abase0 + k * _HW + loff, _ROWS_PER)],
                        avw.at[pl.ds(k * _ROWS_PER, _ROWS_PER)])

    idx_s = [[ia.at[s], ib.at[s], ic.at[s], id_.at[s]] for s in range(2)]
    w_s = [[wa.at[s], wb.at[s], wc.at[s], wd.at[s]] for s in range(2)]
    rows_s = [[ra.at[s], rb.at[s], rc.at[s], rd.at[s]] for s in range(2)]
    sems = [sem0, sem1]

    def stage_and_fire(ci, k, s):
        lo = ci * _CHUNK
        off = loff + lo
        yrow0 = off // _W
        xb = off - yrow0 * _W
        for j in range(_CHUNK // _L):
            sl = pl.ds(j * _L, _L)
            lsl = pl.ds(k * _ROWS_PER + lo + j * _L, _L)
            xc_raw = xb + j * _L + lax.iota(jnp.int32, _L)
            wrap = xc_raw >= _W
            xc = jnp.where(wrap, xc_raw - _W, xc_raw)
            yc = yrow0 + jnp.where(wrap, 1, 0)
            x = jnp.clip(xc.astype(jnp.float32) + fxw[lsl], 0.0, _W - 1.0)
            y = jnp.clip(yc.astype(jnp.float32) + fyw[lsl], 0.0, _H - 1.0)
            x0 = x.astype(jnp.int32)
            y0 = y.astype(jnp.int32)
            dx = x - x0.astype(jnp.float32)
            dy = y - y0.astype(jnp.float32)
            x1 = jnp.minimum(x0 + 1, _W - 1)
            y1 = jnp.minimum(y0 + 1, _H - 1)
            ry0 = imgbase + y0 * _W
            ry1 = imgbase + y1 * _W
            idx_s[s][0][sl] = ry0 + x0
            idx_s[s][1][sl] = ry1 + x0
            idx_s[s][2][sl] = ry0 + x1
            idx_s[s][3][sl] = ry1 + x1
            a_v = avw[lsl]
            omdx = 1.0 - dx
            omdy = 1.0 - dy
            w_s[s][0][sl] = omdx * omdy * a_v
            w_s[s][1][sl] = omdx * dy * a_v
            w_s[s][2][sl] = dx * omdy * a_v
            w_s[s][3][sl] = dx * dy * a_v
        for t in range(4):
            pltpu.async_copy(inp_hbm.at[idx_s[s][t]], rows_s[s][t], sems[s])

    def drain(s):
        for t in range(4):
            pltpu.make_async_copy(inp_hbm.at[idx_s[s][t]], rows_s[s][t],
                                  sems[s]).wait()

    def fma(s, accumulate):
        def fma_body(q, _):
            for u in range(2):
                p = q * 2 + u
                pv = jnp.full((_L,), p, dtype=jnp.int32)
                ws = [plsc.load_gather(w_s[s][t], [pv]) for t in range(4)]
                for cc in range(_C // _L):
                    cs = pl.ds(cc * _L, _L)
                    contrib = (ws[0] * rows_s[s][0][p, cs] +
                               ws[1] * rows_s[s][1][p, cs] +
                               ws[2] * rows_s[s][2][p, cs] +
                               ws[3] * rows_s[s][3][p, cs])
                    if accumulate:
                        ov[p, cs] = ov[p, cs] + contrib
                    else:
                        ov[p, cs] = contrib
            return _

        lax.fori_loop(0, _CHUNK // 2, fma_body, None)

    stage_and_fire(0, 0, 0)

    def chunk_body(ci, carry):
        stage_and_fire(ci, 1, 1)
        drain(0)
        off = wloc * _ROWS_PER + ci * _CHUNK
        dst = out_hbm.at[pl.ds(imgbase + off, _CHUNK), :]

        @pl.when(ci > 0)
        def _wait_prev_out():
            pltpu.make_async_copy(ov, dst, sem_out).wait()

        fma(0, accumulate=False)
        cin = jnp.minimum(ci + 1, _NCHUNK - 1)
        stage_and_fire(cin, 0, 0)
        drain(1)
        fma(1, accumulate=True)
        pltpu.async_copy(ov, dst, sem_out)
        return carry

    lax.fori_loop(0, _NCHUNK, chunk_body, None)
    drain(0)
    lastdst = out_hbm.at[pl.ds(imgbase + loff + (_NCHUNK - 1) * _CHUNK, _CHUNK), :]
    pltpu.make_async_copy(ov, lastdst, sem_out).wait()


def _warp_sc(inp_t, flow_r, att_r):
    mesh = plsc.VectorSubcoreMesh(core_axis_name="c", subcore_axis_name="s")
    return pl.kernel(
        _body,
        out_type=jax.ShapeDtypeStruct((_N, _C), jnp.float32),
        mesh=mesh,
        compiler_params=pltpu.CompilerParams(needs_layout_passes=False),
        scratch_types=[
            pltpu.VMEM((_K * _ROWS_PER,), jnp.float32),
            pltpu.VMEM((_K * _ROWS_PER,), jnp.float32),
            pltpu.VMEM((_K * _ROWS_PER,), jnp.float32),
            pltpu.VMEM((2, _CHUNK), jnp.int32),
            pltpu.VMEM((2, _CHUNK), jnp.int32),
            pltpu.VMEM((2, _CHUNK), jnp.int32),
            pltpu.VMEM((2, _CHUNK), jnp.int32),
            pltpu.VMEM((2, _CHUNK), jnp.float32),
            pltpu.VMEM((2, _CHUNK), jnp.float32),
            pltpu.VMEM((2, _CHUNK), jnp.float32),
            pltpu.VMEM((2, _CHUNK), jnp.float32),
            pltpu.VMEM((2, _CHUNK, _CP), jnp.float32),
            pltpu.VMEM((2, _CHUNK, _CP), jnp.float32),
            pltpu.VMEM((2, _CHUNK, _CP), jnp.float32),
            pltpu.VMEM((2, _CHUNK, _CP), jnp.float32),
            pltpu.VMEM((_CHUNK, _C), jnp.float32),
            pltpu.SemaphoreType.DMA,
            pltpu.SemaphoreType.DMA,
            pltpu.SemaphoreType.DMA,
        ],
    )(inp_t, flow_r, att_r)


def kernel(input, flow, attention):
    inp_t = jnp.transpose(input, (0, 2, 3, 1)).reshape(_N, _C)
    inp_t = jnp.pad(inp_t, ((0, 0), (0, _CP - _C)))
    flow_r = flow.reshape(_B * 2 * _K * _HW)
    att_r = attention.reshape(_B * _K * _HW)
    out_t = _warp_sc(inp_t, flow_r, att_r)
    eye_out = jnp.broadcast_to(jnp.eye(_C, _C, dtype=jnp.float32), (_B, _C, _C))
    out3 = out_t.reshape(_B, _HW, _C)
    out = lax.dot_general(eye_out, out3, (((2,), (2,)), ((0,), (0,))),
                          precision=lax.Precision.HIGHEST)
    return out.reshape(_B, _C, _H, _W)


# speedup vs baseline: 1.2306x; 1.2306x over previous
import jax
import jax.numpy as jnp
from jax import lax
from jax.experimental import pallas as pl
from jax.experimental.pallas import tpu as pltpu
from jax.experimental.pallas import tpu_sc as plsc

_B, _C, _H, _W, _K = 2, 96, 224, 224, 2
_CP = 128
_HW = _H * _W
_N = _B * _HW
_NC, _NS, _L = 2, 16, 16
_NW = _NC * _NS
_ROWS_PER = _N // _NW
_CHUNK = 64
_NCHUNK = _ROWS_PER // _CHUNK
_NTRIPLE = (_NCHUNK - 1) // 3


def _body(inp_hbm, flow_hbm, att_hbm, out_hbm,
          fxw, fyw, avw, ia, ib, ic, id_, wa, wb, wc, wd,
          ra, rb, rc, rd, ov, sg0, sg1, sg2, sem_out):
    wid = lax.axis_index("s") * _NC + lax.axis_index("c")
    b = wid // 16
    wloc = wid - b * 16
    imgbase = b * _HW
    fbase0 = b * (2 * _K * _HW)
    abase0 = b * (_K * _HW)
    loff = wloc * _ROWS_PER

    for k in range(_K):
        pltpu.sync_copy(flow_hbm.at[pl.ds(fbase0 + 2 * k * _HW + loff, _ROWS_PER)],
                        fxw.at[pl.ds(k * _ROWS_PER, _ROWS_PER)])
        pltpu.sync_copy(flow_hbm.at[pl.ds(fbase0 + (2 * k + 1) * _HW + loff, _ROWS_PER)],
                        fyw.at[pl.ds(k * _ROWS_PER, _ROWS_PER)])
        pltpu.sync_copy(att_hbm.at[pl.ds(abase0 + k * _HW + loff, _ROWS_PER)],
                        avw.at[pl.ds(k * _ROWS_PER, _ROWS_PER)])

    idx_s = [[ia.at[s], ib.at[s], ic.at[s], id_.at[s]] for s in range(3)]
    w_s = [[wa.at[s], wb.at[s], wc.at[s], wd.at[s]] for s in range(3)]
    rows_s = [[ra.at[s], rb.at[s], rc.at[s], rd.at[s]] for s in range(3)]
    sems = [sg0, sg1, sg2]

    def stage_and_fire(ci, k, s):
        lo = ci * _CHUNK
        off = loff + lo
        yrow0 = off // _W
        xb = off - yrow0 * _W
        for j in range(_CHUNK // _L):
            sl = pl.ds(j * _L, _L)
            lsl = pl.ds(k * _ROWS_PER + lo + j * _L, _L)
            xc_raw = xb + j * _L + lax.iota(jnp.int32, _L)
            wrap = xc_raw >= _W
            xc = jnp.where(wrap, xc_raw - _W, xc_raw)
            yc = yrow0 + jnp.where(wrap, 1, 0)
            x = jnp.clip(xc.astype(jnp.float32) + fxw[lsl], 0.0, _W - 1.0)
            y = jnp.clip(yc.astype(jnp.float32) + fyw[lsl], 0.0, _H - 1.0)
            x0 = x.astype(jnp.int32)
            y0 = y.astype(jnp.int32)
            dx = x - x0.astype(jnp.float32)
            dy = y - y0.astype(jnp.float32)
            x1 = jnp.minimum(x0 + 1, _W - 1)
            y1 = jnp.minimum(y0 + 1, _H - 1)
            ry0 = imgbase + y0 * _W
            ry1 = imgbase + y1 * _W
            idx_s[s][0][sl] = ry0 + x0
            idx_s[s][1][sl] = ry1 + x0
            idx_s[s][2][sl] = ry0 + x1
            idx_s[s][3][sl] = ry1 + x1
            a_v = avw[lsl]
            omdx = 1.0 - dx
            omdy = 1.0 - dy
            w_s[s][0][sl] = omdx * omdy * a_v
            w_s[s][1][sl] = omdx * dy * a_v
            w_s[s][2][sl] = dx * omdy * a_v
            w_s[s][3][sl] = dx * dy * a_v
        for t in range(4):
            pltpu.async_copy(inp_hbm.at[idx_s[s][t]], rows_s[s][t], sems[s])

    def drain(s):
        for t in range(4):
            pltpu.make_async_copy(inp_hbm.at[idx_s[s][t]], rows_s[s][t],
                                  sems[s]).wait()

    def fma(s, accumulate):
        def fma_body(q, _):
            for u in range(2):
                p = q * 2 + u
                pv = jnp.full((_L,), p, dtype=jnp.int32)
                ws = [plsc.load_gather(w_s[s][t], [pv]) for t in range(4)]
                for cc in range(_C // _L):
                    cs = pl.ds(cc * _L, _L)
                    contrib = (ws[0] * rows_s[s][0][p, cs] +
                               ws[1] * rows_s[s][1][p, cs] +
                               ws[2] * rows_s[s][2][p, cs] +
                               ws[3] * rows_s[s][3][p, cs])
                    if accumulate:
                        ov[p, cs] = ov[p, cs] + contrib
                    else:
                        ov[p, cs] = contrib
            return _

        lax.fori_loop(0, _CHUNK // 2, fma_body, None)

    def out_dst(ci):
        return out_hbm.at[pl.ds(imgbase + loff + ci * _CHUNK, _CHUNK), :]

    stage_and_fire(0, 0, 0)
    stage_and_fire(0, 1, 1)

    def triple_body(t, carry):
        c0 = 3 * t
        for j in range(6):
            ph = 6 * t + 2 + j
            fci = 3 * t + (2 + j) // 2
            fk = j % 2
            fs = (2 + j) % 3
            stage_and_fire(fci, fk, fs)
            dj = j
            ds_ = dj % 3
            drain(ds_)
            ci = c0 + dj // 2
            if dj % 2 == 0:
                if j == 0:
                    @pl.when(t > 0)
                    def _wait_prev_out():
                        pltpu.make_async_copy(ov, out_dst(ci), sem_out).wait()
                else:
                    pltpu.make_async_copy(ov, out_dst(ci), sem_out).wait()
                fma(ds_, accumulate=False)
            else:
                fma(ds_, accumulate=True)
                pltpu.async_copy(ov, out_dst(ci), sem_out)
        return carry

    lax.fori_loop(0, _NTRIPLE, triple_body, None)

    ci = _NCHUNK - 1
    drain(0)
    pltpu.make_async_copy(ov, out_dst(ci), sem_out).wait()
    fma(0, accumulate=False)
    drain(1)
    fma(1, accumulate=True)
    pltpu.async_copy(ov, out_dst(ci), sem_out)
    pltpu.make_async_copy(ov, out_dst(ci), sem_out).wait()


def _warp_sc(inp_t, flow_r, att_r):
    mesh = plsc.VectorSubcoreMesh(core_axis_name="c", subcore_axis_name="s")
    return pl.kernel(
        _body,
        out_type=jax.ShapeDtypeStruct((_N, _C), jnp.float32),
        mesh=mesh,
        compiler_params=pltpu.CompilerParams(needs_layout_passes=False),
        scratch_types=[
            pltpu.VMEM((_K * _ROWS_PER,), jnp.float32),
            pltpu.VMEM((_K * _ROWS_PER,), jnp.float32),
            pltpu.VMEM((_K * _ROWS_PER,), jnp.float32),
            pltpu.VMEM((3, _CHUNK), jnp.int32),
            pltpu.VMEM((3, _CHUNK), jnp.int32),
            pltpu.VMEM((3, _CHUNK), jnp.int32),
            pltpu.VMEM((3, _CHUNK), jnp.int32),
            pltpu.VMEM((3, _CHUNK), jnp.float32),
            pltpu.VMEM((3, _CHUNK), jnp.float32),
            pltpu.VMEM((3, _CHUNK), jnp.float32),
            pltpu.VMEM((3, _CHUNK), jnp.float32),
            pltpu.VMEM((3, _CHUNK, _CP), jnp.float32),
            pltpu.VMEM((3, _CHUNK, _CP), jnp.float32),
            pltpu.VMEM((3, _CHUNK, _CP), jnp.float32),
            pltpu.VMEM((3, _CHUNK, _CP), jnp.float32),
            pltpu.VMEM((_CHUNK, _C), jnp.float32),
            pltpu.SemaphoreType.DMA,
            pltpu.SemaphoreType.DMA,
            pltpu.SemaphoreType.DMA,
            pltpu.SemaphoreType.DMA,
        ],
    )(inp_t, flow_r, att_r)


def kernel(input, flow, attention):
    inp_t = jnp.transpose(input, (0, 2, 3, 1)).reshape(_N, _C)
    inp_t = jnp.pad(inp_t, ((0, 0), (0, _CP - _C)))
    flow_r = flow.reshape(_B * 2 * _K * _HW)
    att_r = attention.reshape(_B * _K * _HW)
    out_t = _warp_sc(inp_t, flow_r, att_r)
    return jnp.transpose(out_t.reshape(_B, _H, _W, _C), (0, 3, 1, 2))
